# baseline (speedup 1.0000x reference)
import functools

import jax
import jax.numpy as jnp
from jax import lax
from jax.experimental import pallas as pl
from jax.experimental.pallas import tpu as pltpu
from jax.experimental.pallas import tpu_sc as plsc

NC = 2
NS = 16
NW = NC * NS
C = 128
D = 64


@functools.partial(jax.jit, static_argnums=(2,))
def _gather(table2, idx, n_total):
    n_per_w = n_total // NW
    nch = n_per_w // C
    assert nch % 2 == 0
    mesh = plsc.VectorSubcoreMesh(core_axis_name="c", subcore_axis_name="s")

    @functools.partial(
        pl.kernel,
        mesh=mesh,
        out_type=jax.ShapeDtypeStruct((n_total // 2, 2 * D), jnp.float32),
        scratch_types=[
            pltpu.VMEM((nch, C), jnp.int32),
            pltpu.VMEM((nch, C), jnp.int32),
            pltpu.VMEM((2, C, 2 * D), jnp.float32),
            pltpu.VMEM((2, C // 2, 2 * D), jnp.float32),
            pltpu.SemaphoreType.DMA,
            pltpu.SemaphoreType.DMA,
            pltpu.SemaphoreType.DMA,
            pltpu.SemaphoreType.DMA,
        ],
        compiler_params=pltpu.CompilerParams(use_tc_tiling_on_sc=True, needs_layout_passes=False),
    )
    def gather_kernel(table_hbm, idx_hbm, out_hbm, idx_v, idx2_v, buf_v,
                      stage_v, gsem0, gsem1, ssem0, ssem1):
        sid = lax.axis_index("s")
        wid = sid * NC + lax.axis_index("c")
        base = wid * n_per_w
        qbase = base // 2
        gsems = (gsem0, gsem1)
        ssems = (ssem0, ssem1)
        pltpu.sync_copy(idx_hbm.at[wid], idx_v)

        lane = lax.iota(jnp.int32, 16)

        @pl.loop(0, nch)
        def _prep(j):
            for g in range(C // 16):
                v = idx_v[j, pl.ds(g * 16, 16)]
                idx2_v[j, pl.ds(g * 16, 16)] = v >> 1

        pltpu.async_copy(table_hbm.at[idx2_v.at[0]], buf_v.at[0], gsem0)
        pltpu.async_copy(table_hbm.at[idx2_v.at[1]], buf_v.at[1], gsem1)

        def select_pack(j, p):
            for g in range(C // 16):
                v = idx_v[j, pl.ds(g * 16, 16)]
                half = (v & 1) * D
                row = g * 16 + lane
                qrow = row >> 1
                qcol0 = (row & 1) * D

                @pl.loop(0, D)
                def _c(c):
                    vals = plsc.load_gather(buf_v.at[p], [row, half + c])
                    plsc.store_scatter(stage_v.at[p], [qrow, qcol0 + c], vals)

        @pl.loop(0, nch, step=2)
        def _chunk(j0):
            for p in range(2):
                j = j0 + p
                pltpu.make_async_copy(
                    table_hbm.at[idx2_v.at[j]], buf_v.at[p], gsems[p]
                ).wait()

                @pl.when(j >= 2)
                def _():
                    pltpu.make_async_copy(
                        stage_v.at[p], out_hbm.at[pl.ds(pl.multiple_of(qbase, 8), C // 2)], ssems[p]
                    ).wait()

                select_pack(j, p)

                @pl.when(j + 2 < nch)
                def _():
                    pltpu.async_copy(
                        table_hbm.at[idx2_v.at[j + 2]], buf_v.at[p], gsems[p]
                    )

                pltpu.async_copy(
                    stage_v.at[p],
                    out_hbm.at[pl.ds(pl.multiple_of(qbase + j * (C // 2), 8), C // 2)],
                    ssems[p],
                )

        for p in range(2):
            pltpu.make_async_copy(
                stage_v.at[p], out_hbm.at[pl.ds(pl.multiple_of(qbase, 8), C // 2)], ssems[p]
            ).wait()

    return gather_kernel(table2, idx)


def kernel(x, embed):
    b, h = x.shape
    n_total = b * h
    table2 = embed.reshape(embed.shape[0] // 2, 2 * D)
    idx = x.astype(jnp.int32).reshape(NW, n_total // (NW * C), C)
    out2 = _gather(table2, idx, n_total)
    return out2.reshape(b, h, D)
